# baseline (speedup 1.0000x reference)
import functools

import jax
import jax.numpy as jnp
from jax.experimental import pallas as pl
from jax.experimental.pallas import tpu as pltpu

_UNROLL = 8


def _tree_min(parts):
    while len(parts) > 1:
        nxt = [jnp.minimum(parts[k], parts[k + 1]) for k in range(0, len(parts) - 1, 2)]
        if len(parts) % 2:
            nxt.append(parts[-1])
        parts = nxt
    return parts[0]


def _chamfer_body(a1_ref, a2_ref, out_ref, pmin1_ref, pmin2_ref, *, batches):
    i = pl.program_id(0)
    b = batches
    u = _UNROLL

    def tail_sums():
        total = 0.0
        for k in range(u):
            pm1 = pmin1_ref[k]
            pm2 = pmin2_ref[k]
            n = pm1.shape[0]
            m = pm2.shape[1]
            s1 = jnp.sum(jnp.min(pm1, axis=1)) * (1.0 / (b * n))
            s2 = jnp.sum(jnp.min(pm2, axis=0)) * (1.0 / (b * m))
            total = total + (s1 + s2)
        return total

    contrib = jnp.where(i > 0, tail_sums(), 0.0)
    acc = jnp.where(i > 0, out_ref[0, 0], 0.0)
    out_ref[0, 0] = acc + contrib

    def rows5(ref, bi, scale):
        x = ref[pl.ds(bi, 1), :]
        y = ref[pl.ds(bi + b, 1), :]
        z = ref[pl.ds(bi + 2 * b, 1), :]
        sq = x * x + y * y + z * z
        ones = jnp.ones_like(x)
        if scale is None:
            return jnp.concatenate([x, y, z, sq, ones], axis=0)
        return jnp.concatenate(
            [scale * x, scale * y, scale * z, ones, sq], axis=0
        )

    for k in range(u):
        bi = i * u + k
        lhs = rows5(a1_ref, bi, None)
        rhs = rows5(a2_ref, bi, -2.0)
        d = jax.lax.dot_general(
            lhs, rhs,
            dimension_numbers=(((0,), (0,)), ((), ())),
            preferred_element_type=jnp.float32,
        )
        m = d.shape[1]
        pmin1_ref[k] = _tree_min(
            [d[:, c * 128:(c + 1) * 128] for c in range(m // 128)]
        )
        pmin2_ref[k] = _tree_min(
            [d[r * 8:(r + 1) * 8, :] for r in range(d.shape[0] // 8)]
        )

    @pl.when(i == (b // u) - 1)
    def _last():
        out_ref[0, 0] += tail_sums()


@functools.partial(jax.jit, static_argnames=("interpret",))
def _chamfer(array1, array2, interpret=False):
    b, n, _ = array1.shape
    m = array2.shape[1]
    a1v = array1.transpose(2, 0, 1).reshape(3 * b, n)
    a2v = array2.transpose(2, 0, 1).reshape(3 * b, m)
    out = pl.pallas_call(
        functools.partial(_chamfer_body, batches=b),
        grid=(b // _UNROLL,),
        in_specs=[
            pl.BlockSpec((3 * b, n), lambda i: (0, 0)),
            pl.BlockSpec((3 * b, m), lambda i: (0, 0)),
        ],
        out_specs=pl.BlockSpec(
            (1, 1), lambda i: (0, 0), memory_space=pltpu.SMEM
        ),
        out_shape=jax.ShapeDtypeStruct((1, 1), jnp.float32),
        scratch_shapes=[
            pltpu.VMEM((_UNROLL, n, 128), jnp.float32),
            pltpu.VMEM((_UNROLL, 8, m), jnp.float32),
        ],
        interpret=interpret,
    )(a1v, a2v)
    return out.reshape(())


def kernel(array1, array2):
    return _chamfer(array1, array2)


# speedup vs baseline: 1.0205x; 1.0205x over previous
import functools

import jax
import jax.numpy as jnp
from jax.experimental import pallas as pl
from jax.experimental.pallas import tpu as pltpu

_UNROLL = 4


def _tree_min(parts):
    while len(parts) > 1:
        nxt = [jnp.minimum(parts[k], parts[k + 1]) for k in range(0, len(parts) - 1, 2)]
        if len(parts) % 2:
            nxt.append(parts[-1])
        parts = nxt
    return parts[0]


def _chamfer_body(a1_ref, a2_ref, out_ref, pmin1_ref, pmin2_ref, *, batches):
    i = pl.program_id(0)
    b = batches
    u = _UNROLL

    def tail_sums():
        total = 0.0
        for k in range(u):
            pm1 = pmin1_ref[k]
            pm2 = pmin2_ref[k]
            n = pm1.shape[0]
            m = pm2.shape[1]
            s1 = jnp.sum(jnp.min(pm1, axis=1)) * (1.0 / (b * n))
            s2 = jnp.sum(jnp.min(pm2, axis=0)) * (1.0 / (b * m))
            total = total + (s1 + s2)
        return total

    contrib = jnp.where(i > 0, tail_sums(), 0.0)
    acc = jnp.where(i > 0, out_ref[0, 0], 0.0)
    out_ref[0, 0] = acc + contrib

    def rows5(ref, bi, scale):
        x = ref[pl.ds(bi, 1), :]
        y = ref[pl.ds(bi + b, 1), :]
        z = ref[pl.ds(bi + 2 * b, 1), :]
        sq = x * x + y * y + z * z
        ones = jnp.ones_like(x)
        if scale is None:
            return jnp.concatenate([x, y, z, sq, ones], axis=0)
        return jnp.concatenate(
            [scale * x, scale * y, scale * z, ones, sq], axis=0
        )

    for k in range(u):
        bi = i * u + k
        lhs = rows5(a1_ref, bi, None)
        rhs = rows5(a2_ref, bi, -2.0)
        d = jax.lax.dot_general(
            lhs, rhs,
            dimension_numbers=(((0,), (0,)), ((), ())),
            preferred_element_type=jnp.float32,
        )
        m = d.shape[1]
        pmin1_ref[k] = _tree_min(
            [d[:, c * 128:(c + 1) * 128] for c in range(m // 128)]
        )
        pmin2_ref[k] = _tree_min(
            [d[r * 8:(r + 1) * 8, :] for r in range(d.shape[0] // 8)]
        )

    @pl.when(i == (b // u) - 1)
    def _last():
        out_ref[0, 0] += tail_sums()


@functools.partial(jax.jit, static_argnames=("interpret",))
def _chamfer(array1, array2, interpret=False):
    b, n, _ = array1.shape
    m = array2.shape[1]
    a1v = array1.transpose(2, 0, 1).reshape(3 * b, n)
    a2v = array2.transpose(2, 0, 1).reshape(3 * b, m)
    out = pl.pallas_call(
        functools.partial(_chamfer_body, batches=b),
        grid=(b // _UNROLL,),
        in_specs=[
            pl.BlockSpec((3 * b, n), lambda i: (0, 0)),
            pl.BlockSpec((3 * b, m), lambda i: (0, 0)),
        ],
        out_specs=pl.BlockSpec(
            (1, 1), lambda i: (0, 0), memory_space=pltpu.SMEM
        ),
        out_shape=jax.ShapeDtypeStruct((1, 1), jnp.float32),
        scratch_shapes=[
            pltpu.VMEM((_UNROLL, n, 128), jnp.float32),
            pltpu.VMEM((_UNROLL, 8, m), jnp.float32),
        ],
        interpret=interpret,
    )(a1v, a2v)
    return out.reshape(())


def kernel(array1, array2):
    return _chamfer(array1, array2)
